# baseline (speedup 1.0000x reference)
import dataclasses
import functools

import jax
import jax.numpy as jnp
from jax import lax
from jax.experimental import pallas as pl
from jax.experimental.pallas import tpu as pltpu
from jax.experimental.pallas import tpu_sc as plsc

MAX_POS = 1024
HIDDEN = 768
HW = HIDDEN // 2
EPS = 1e-12

NC = 2
NS = 16
NW = NC * NS

NTOK = 4 * 2048
NH = 2
NTOK_H = NTOK // NH
T_CH = 32
ROWS = 4 * T_CH
NB = 2


def _sc_gather_sum(px, py, idx0, idx1, idx2, idx3):
    ntok = idx0.shape[0]
    TPW = ntok // NW
    NCH = TPW // T_CH
    mesh = plsc.VectorSubcoreMesh(core_axis_name="c", subcore_axis_name="s")
    cp = pltpu.CompilerParams()
    if "needs_layout_passes" in pltpu.CompilerParams.__dataclass_fields__:
        cp = dataclasses.replace(cp, needs_layout_passes=False)

    @functools.partial(
        pl.kernel,
        mesh=mesh,
        compiler_params=cp,
        out_type=jax.ShapeDtypeStruct((ntok, HW), jnp.int32),
        scratch_types=[
            pltpu.VMEM((4 * TPW,), jnp.int32),
            pltpu.VMEM((ROWS, HW), jnp.int32),
            pltpu.VMEM((ROWS, HW), jnp.int32),
            pltpu.VMEM((T_CH, HW), jnp.int32),
            pltpu.VMEM((T_CH, HW), jnp.int32),
            pltpu.SemaphoreType.DMA,
            pltpu.SemaphoreType.DMA,
            pltpu.SemaphoreType.DMA,
            pltpu.SemaphoreType.DMA,
        ],
    )
    def k(px_hbm, py_hbm, i0_hbm, i1_hbm, i2_hbm, i3_hbm, out_hbm,
          idx_v, rows0, rows1, acc0, acc1, sg0, sg1, sw0, sw1):
        wid = lax.axis_index("s") * NC + lax.axis_index("c")
        base = wid * TPW
        for kk, ih in enumerate([i0_hbm, i1_hbm, i2_hbm, i3_hbm]):
            pltpu.sync_copy(ih.at[pl.ds(base, TPW)],
                            idx_v.at[pl.ds(kk * TPW, TPW)])

        rows = [rows0, rows1]
        acc = [acc0, acc1]
        sg = [sg0, sg1]
        sw = [sw0, sw1]
        tables = [px_hbm, py_hbm, px_hbm, py_hbm]

        def start_gather(c):
            b = c % NB
            return [
                pltpu.async_copy(
                    tables[kk].at[idx_v.at[pl.ds(kk * TPW + c * T_CH, T_CH)]],
                    rows[b].at[pl.ds(kk * T_CH, T_CH)], sg[b])
                for kk in range(4)
            ]

        def accum(c):
            rv, av = rows[c % NB], acc[c % NB]

            @pl.loop(0, HW // 16)
            def _grp(g):
                sl = pl.ds(g * 16, 16)
                for t in range(T_CH):
                    v0 = plsc.bitcast(rv[t, sl], jnp.bfloat16)
                    v1 = plsc.bitcast(rv[T_CH + t, sl], jnp.bfloat16)
                    v2 = plsc.bitcast(rv[2 * T_CH + t, sl], jnp.bfloat16)
                    v3 = plsc.bitcast(rv[3 * T_CH + t, sl], jnp.bfloat16)
                    av[t, sl] = plsc.bitcast((v0 + v1) + (v2 + v3), jnp.int32)

        gh = [None] * NCH
        wh = [None] * NCH
        gh[0] = start_gather(0)
        for c in range(NCH):
            if c + 1 < NCH:
                gh[c + 1] = start_gather(c + 1)
            for h in gh[c]:
                h.wait()
            if c >= NB:
                wh[c - NB].wait()
            accum(c)
            wh[c] = pltpu.async_copy(
                acc[c % NB], out_hbm.at[pl.ds(base + c * T_CH, T_CH)],
                sw[c % NB])
        for c in range(NCH - NB, NCH):
            wh[c].wait()

    return k(px, py, idx0, idx1, idx2, idx3)


BT = 2048
NBLK_H = NTOK_H // BT


def _tc_ln_mlp_half(emb_i32, w_g, bias_eff, h, prev):
    def body(*refs):
        if h == 0:
            emb_ref, w_ref, bias_ref, o_ref = refs
        else:
            _, emb_ref, w_ref, bias_ref, o_ref = refs
        xi = emb_ref[...]
        ev = lax.bitcast_convert_type(xi << 16, jnp.float32)
        od = lax.bitcast_convert_type(xi & jnp.int32(-65536), jnp.float32)
        x = jnp.concatenate([ev, od], axis=1)
        mean = jnp.mean(x, axis=1, keepdims=True)
        msq = jnp.mean(x * x, axis=1, keepdims=True)
        rs = lax.rsqrt(msq - mean * mean + EPS)
        xn = (x * rs - mean * rs).astype(jnp.bfloat16)
        y = lax.dot_general(
            xn,
            w_ref[...],
            (((1,), (1,)), ((), ())),
            preferred_element_type=jnp.float32,
        )
        o_ref[...] = y + bias_ref[...]

    in_specs = [
        pl.BlockSpec((BT, HW), lambda i: (i, 0)),
        pl.BlockSpec((HIDDEN, HIDDEN), lambda i: (0, 0)),
        pl.BlockSpec((1, HIDDEN), lambda i: (0, 0)),
    ]
    args = [emb_i32, w_g, bias_eff]
    io_aliases = {}
    if h > 0:
        in_specs = [pl.BlockSpec(memory_space=pl.ANY)] + in_specs
        args = [prev] + args
        io_aliases = {0: 0}

    return pl.pallas_call(
        body,
        grid=(NBLK_H,),
        in_specs=in_specs,
        out_specs=pl.BlockSpec((BT, HIDDEN), lambda i, _h=h: (i + _h * NBLK_H, 0)),
        out_shape=jax.ShapeDtypeStruct((NTOK, HIDDEN), jnp.float32),
        input_output_aliases=io_aliases,
    )(*args)


def kernel(bbox, x_table, y_table, ln_gamma, ln_beta, W, b):
    def pack(t):
        bits = lax.bitcast_convert_type(t, jnp.uint32) + jnp.uint32(0x8000)
        return lax.bitcast_convert_type(
            (bits[:, :HW] >> 16) | (bits[:, HW:] & jnp.uint32(0xFFFF0000)),
            jnp.int32)

    bb = bbox.astype(jnp.int32)
    idx = [bb[:, :, kk].reshape(-1) for kk in range(4)]
    px, py = pack(x_table), pack(y_table)
    w_g = (W * ln_gamma[None, :]).astype(jnp.bfloat16)
    bias_eff = (b + ln_beta @ W.T).reshape(1, HIDDEN)

    embs = []
    for h in range(NH):
        sl = slice(h * NTOK_H, (h + 1) * NTOK_H)
        embs.append(_sc_gather_sum(px, py, *[ix[sl] for ix in idx]))
    out = None
    for h in range(NH):
        out = _tc_ln_mlp_half(embs[h], w_g, bias_eff, h, out)
    return out.reshape(bbox.shape[0], bbox.shape[1], HIDDEN)


# speedup vs baseline: 1.0176x; 1.0176x over previous
import dataclasses
import functools

import jax
import jax.numpy as jnp
from jax import lax
from jax.experimental import pallas as pl
from jax.experimental.pallas import tpu as pltpu
from jax.experimental.pallas import tpu_sc as plsc

MAX_POS = 1024
HIDDEN = 768
HW = HIDDEN // 2
EPS = 1e-12

NC = 2
NS = 16
NW = NC * NS

NTOK = 4 * 2048
PARTS = (6144, 2048)
OFFS = (0, 6144)
T_CH = 32
ROWS = 4 * T_CH
NB = 2


def _sc_gather_sum(px, py, idx0, idx1, idx2, idx3):
    ntok = idx0.shape[0]
    TPW = ntok // NW
    NCH = TPW // T_CH
    mesh = plsc.VectorSubcoreMesh(core_axis_name="c", subcore_axis_name="s")
    cp = pltpu.CompilerParams()
    if "needs_layout_passes" in pltpu.CompilerParams.__dataclass_fields__:
        cp = dataclasses.replace(cp, needs_layout_passes=False)

    @functools.partial(
        pl.kernel,
        mesh=mesh,
        compiler_params=cp,
        out_type=jax.ShapeDtypeStruct((ntok, HW), jnp.int32),
        scratch_types=[
            pltpu.VMEM((4 * TPW,), jnp.int32),
            pltpu.VMEM((ROWS, HW), jnp.int32),
            pltpu.VMEM((ROWS, HW), jnp.int32),
            pltpu.VMEM((T_CH, HW), jnp.int32),
            pltpu.VMEM((T_CH, HW), jnp.int32),
            pltpu.SemaphoreType.DMA,
            pltpu.SemaphoreType.DMA,
            pltpu.SemaphoreType.DMA,
            pltpu.SemaphoreType.DMA,
        ],
    )
    def k(px_hbm, py_hbm, i0_hbm, i1_hbm, i2_hbm, i3_hbm, out_hbm,
          idx_v, rows0, rows1, acc0, acc1, sg0, sg1, sw0, sw1):
        wid = lax.axis_index("s") * NC + lax.axis_index("c")
        base = wid * TPW
        for kk, ih in enumerate([i0_hbm, i1_hbm, i2_hbm, i3_hbm]):
            pltpu.sync_copy(ih.at[pl.ds(base, TPW)],
                            idx_v.at[pl.ds(kk * TPW, TPW)])

        rows = [rows0, rows1]
        acc = [acc0, acc1]
        sg = [sg0, sg1]
        sw = [sw0, sw1]
        tables = [px_hbm, py_hbm, px_hbm, py_hbm]

        def start_gather(c):
            b = c % NB
            return [
                pltpu.async_copy(
                    tables[kk].at[idx_v.at[pl.ds(kk * TPW + c * T_CH, T_CH)]],
                    rows[b].at[pl.ds(kk * T_CH, T_CH)], sg[b])
                for kk in range(4)
            ]

        def accum(c):
            rv, av = rows[c % NB], acc[c % NB]

            @pl.loop(0, HW // 16)
            def _grp(g):
                sl = pl.ds(g * 16, 16)
                for t in range(T_CH):
                    v0 = plsc.bitcast(rv[t, sl], jnp.bfloat16)
                    v1 = plsc.bitcast(rv[T_CH + t, sl], jnp.bfloat16)
                    v2 = plsc.bitcast(rv[2 * T_CH + t, sl], jnp.bfloat16)
                    v3 = plsc.bitcast(rv[3 * T_CH + t, sl], jnp.bfloat16)
                    av[t, sl] = plsc.bitcast((v0 + v1) + (v2 + v3), jnp.int32)

        gh = [None] * NCH
        wh = [None] * NCH
        gh[0] = start_gather(0)
        for c in range(NCH):
            if c + 1 < NCH:
                gh[c + 1] = start_gather(c + 1)
            for h in gh[c]:
                h.wait()
            if c >= NB:
                wh[c - NB].wait()
            accum(c)
            wh[c] = pltpu.async_copy(
                acc[c % NB], out_hbm.at[pl.ds(base + c * T_CH, T_CH)],
                sw[c % NB])
        for c in range(NCH - NB, NCH):
            wh[c].wait()

    return k(px, py, idx0, idx1, idx2, idx3)


BT = 2048


def _tc_ln_mlp_part(emb_i32, w_g, bias_eff, h, off, prev):
    ntok_h = emb_i32.shape[0]
    nblk = ntok_h // BT
    blk0 = off // BT

    def body(*refs):
        if h == 0:
            emb_ref, w_ref, bias_ref, o_ref = refs
        else:
            _, emb_ref, w_ref, bias_ref, o_ref = refs
        xi = emb_ref[...]
        ev = lax.bitcast_convert_type(xi << 16, jnp.float32)
        od = lax.bitcast_convert_type(xi & jnp.int32(-65536), jnp.float32)
        x = jnp.concatenate([ev, od], axis=1)
        mean = jnp.mean(x, axis=1, keepdims=True)
        msq = jnp.mean(x * x, axis=1, keepdims=True)
        rs = lax.rsqrt(msq - mean * mean + EPS)
        xn = (x * rs - mean * rs).astype(jnp.bfloat16)
        y = lax.dot_general(
            xn,
            w_ref[...],
            (((1,), (1,)), ((), ())),
            preferred_element_type=jnp.float32,
        )
        o_ref[...] = y + bias_ref[...]

    in_specs = [
        pl.BlockSpec((BT, HW), lambda i: (i, 0)),
        pl.BlockSpec((HIDDEN, HIDDEN), lambda i: (0, 0)),
        pl.BlockSpec((1, HIDDEN), lambda i: (0, 0)),
    ]
    args = [emb_i32, w_g, bias_eff]
    io_aliases = {}
    if h > 0:
        in_specs = [pl.BlockSpec(memory_space=pl.ANY)] + in_specs
        args = [prev] + args
        io_aliases = {0: 0}

    return pl.pallas_call(
        body,
        grid=(nblk,),
        in_specs=in_specs,
        out_specs=pl.BlockSpec((BT, HIDDEN), lambda i: (i + blk0, 0)),
        out_shape=jax.ShapeDtypeStruct((NTOK, HIDDEN), jnp.float32),
        input_output_aliases=io_aliases,
    )(*args)


def kernel(bbox, x_table, y_table, ln_gamma, ln_beta, W, b):
    def pack(t):
        bits = lax.bitcast_convert_type(t, jnp.uint32) + jnp.uint32(0x8000)
        return lax.bitcast_convert_type(
            (bits[:, :HW] >> 16) | (bits[:, HW:] & jnp.uint32(0xFFFF0000)),
            jnp.int32)

    bb = bbox.astype(jnp.int32)
    idx = [bb[:, :, kk].reshape(-1) for kk in range(4)]
    px, py = pack(x_table), pack(y_table)
    w_g = (W * ln_gamma[None, :]).astype(jnp.bfloat16)
    bias_eff = (b + ln_beta @ W.T).reshape(1, HIDDEN)

    embs = []
    for h, (off, n) in enumerate(zip(OFFS, PARTS)):
        sl = slice(off, off + n)
        embs.append(_sc_gather_sum(px, py, *[ix[sl] for ix in idx]))
    out = None
    for h, off in enumerate(OFFS):
        out = _tc_ln_mlp_part(embs[h], w_g, bias_eff, h, off, out)
    return out.reshape(bbox.shape[0], bbox.shape[1], HIDDEN)


# speedup vs baseline: 1.0873x; 1.0685x over previous
import dataclasses
import functools

import jax
import jax.numpy as jnp
from jax import lax
from jax.experimental import pallas as pl
from jax.experimental.pallas import tpu as pltpu
from jax.experimental.pallas import tpu_sc as plsc

MAX_POS = 1024
HIDDEN = 768
HW = HIDDEN // 2
EPS = 1e-12

NC = 2
NS = 16
NW = NC * NS

NTOK = 4 * 2048
TPW = NTOK // NW
T_CH = 32
ROWS = 4 * T_CH
NCH = TPW // T_CH
NB = 2


def _sc_gather_sum(px, py, idx0, idx1, idx2, idx3):
    mesh = plsc.VectorSubcoreMesh(core_axis_name="c", subcore_axis_name="s")
    cp = pltpu.CompilerParams()
    if "needs_layout_passes" in pltpu.CompilerParams.__dataclass_fields__:
        cp = dataclasses.replace(cp, needs_layout_passes=False)

    @functools.partial(
        pl.kernel,
        mesh=mesh,
        compiler_params=cp,
        out_type=jax.ShapeDtypeStruct((NTOK, HW), jnp.int32),
        scratch_types=[
            pltpu.VMEM((4 * TPW,), jnp.int32),
            pltpu.VMEM((ROWS, HW), jnp.int32),
            pltpu.VMEM((ROWS, HW), jnp.int32),
            pltpu.VMEM((T_CH, HW), jnp.int32),
            pltpu.VMEM((T_CH, HW), jnp.int32),
            pltpu.SemaphoreType.DMA,
            pltpu.SemaphoreType.DMA,
            pltpu.SemaphoreType.DMA,
            pltpu.SemaphoreType.DMA,
        ],
    )
    def k(px_hbm, py_hbm, i0_hbm, i1_hbm, i2_hbm, i3_hbm, out_hbm,
          idx_v, rows0, rows1, acc0, acc1, sg0, sg1, sw0, sw1):
        wid = lax.axis_index("s") * NC + lax.axis_index("c")
        base = wid * TPW
        for kk, ih in enumerate([i0_hbm, i1_hbm, i2_hbm, i3_hbm]):
            pltpu.sync_copy(ih.at[pl.ds(base, TPW)],
                            idx_v.at[pl.ds(kk * TPW, TPW)])

        rows = [rows0, rows1]
        acc = [acc0, acc1]
        sg = [sg0, sg1]
        sw = [sw0, sw1]
        tables = [px_hbm, py_hbm, px_hbm, py_hbm]

        def start_gather(c):
            b = c % NB
            return [
                pltpu.async_copy(
                    tables[kk].at[idx_v.at[pl.ds(kk * TPW + c * T_CH, T_CH)]],
                    rows[b].at[pl.ds(kk * T_CH, T_CH)], sg[b])
                for kk in range(4)
            ]

        def accum(c):
            rv, av = rows[c % NB], acc[c % NB]

            @pl.loop(0, HW // 16)
            def _grp(g):
                sl = pl.ds(g * 16, 16)
                for t in range(T_CH):
                    v0 = plsc.bitcast(rv[t, sl], jnp.bfloat16)
                    v1 = plsc.bitcast(rv[T_CH + t, sl], jnp.bfloat16)
                    v2 = plsc.bitcast(rv[2 * T_CH + t, sl], jnp.bfloat16)
                    v3 = plsc.bitcast(rv[3 * T_CH + t, sl], jnp.bfloat16)
                    av[t, sl] = plsc.bitcast((v0 + v1) + (v2 + v3), jnp.int32)

        gh = [None] * NCH
        wh = [None] * NCH
        gh[0] = start_gather(0)
        for c in range(NCH):
            if c + 1 < NCH:
                gh[c + 1] = start_gather(c + 1)
            for h in gh[c]:
                h.wait()
            if c >= NB:
                wh[c - NB].wait()
            accum(c)
            wh[c] = pltpu.async_copy(
                acc[c % NB], out_hbm.at[pl.ds(base + c * T_CH, T_CH)],
                sw[c % NB])
        for c in range(NCH - NB, NCH):
            wh[c].wait()

    return k(px, py, idx0, idx1, idx2, idx3)


BT = 2048


def _tc_ln_mlp(emb_i32, gamma, beta, W, b):
    w_g = (W * gamma[None, :]).astype(jnp.bfloat16)
    bias_eff = b + beta @ W.T

    def body(emb_ref, w_ref, bias_ref, o_ref):
        xi = emb_ref[...]
        ev = lax.bitcast_convert_type(xi << 16, jnp.float32)
        od = lax.bitcast_convert_type(xi & jnp.int32(-65536), jnp.float32)
        x = jnp.concatenate([ev, od], axis=1)
        mean = jnp.mean(x, axis=1, keepdims=True)
        msq = jnp.mean(x * x, axis=1, keepdims=True)
        rs = lax.rsqrt(msq - mean * mean + EPS)
        xn = (x * rs - mean * rs).astype(jnp.bfloat16)
        y = lax.dot_general(
            xn,
            w_ref[...],
            (((1,), (1,)), ((), ())),
            preferred_element_type=jnp.float32,
        )
        o_ref[...] = y + bias_ref[...]

    return pl.pallas_call(
        body,
        grid=(NTOK // BT,),
        in_specs=[
            pl.BlockSpec((BT, HW), lambda i: (i, 0)),
            pl.BlockSpec((HIDDEN, HIDDEN), lambda i: (0, 0)),
            pl.BlockSpec((1, HIDDEN), lambda i: (0, 0)),
        ],
        out_specs=pl.BlockSpec((BT, HIDDEN), lambda i: (i, 0)),
        out_shape=jax.ShapeDtypeStruct((NTOK, HIDDEN), jnp.float32),
    )(
        emb_i32,
        w_g,
        bias_eff.reshape(1, HIDDEN),
    )


def kernel(bbox, x_table, y_table, ln_gamma, ln_beta, W, b):
    def pack(t):
        bits = lax.bitcast_convert_type(t, jnp.uint32) + jnp.uint32(0x8000)
        return lax.bitcast_convert_type(
            (bits[:, :HW] >> 16) | (bits[:, HW:] & jnp.uint32(0xFFFF0000)),
            jnp.int32)

    bb = bbox.astype(jnp.int32)
    idx = [bb[:, :, kk].reshape(-1) for kk in range(4)]
    emb_i32 = _sc_gather_sum(pack(x_table), pack(y_table), *idx)
    out = _tc_ln_mlp(emb_i32, ln_gamma, ln_beta, W, b)
    return out.reshape(bbox.shape[0], bbox.shape[1], HIDDEN)
